# baseline (speedup 1.0000x reference)
import numpy as np
import jax
import jax.numpy as jnp
from jax import lax
from jax.experimental import pallas as pl
from jax.experimental.pallas import tpu as pltpu

B = 2; T = 2048; V = 8; P = 64; NPATCH = T // P; C = 768; H = 12; DH = C // H
LAYERS = 2; NEXP = 8; TOPK = 2; DFF = 3072; K = 10; L = NPATCH + 1
FREQ_P = P // 2 + 1; FREQ_S = T // 2 + 1
NSEQ = B * V
LP = 40
NTOK = NSEQ * LP
NCH = 2
HC = DFF // NCH
NBUF = 2
NUNIT = LAYERS * NEXP * NCH
CAP = 256
SLOT = NEXP * CAP


def _dft_mats(n, nfreq):
    ns = np.arange(n)[:, None]
    ks = np.arange(nfreq)[None, :]
    ang = 2.0 * np.pi * ns * ks / n
    return np.cos(ang).astype(np.float32), np.sin(ang).astype(np.float32)

_DPC, _DPS = _dft_mats(P, FREQ_P)
_DSC, _DSS = _dft_mats(T, FREQ_S)
_SU128 = np.triu(np.ones((128, 128), np.float32), 1)
_SL640 = np.tril(np.ones((NTOK, NTOK), np.float32), -1)


def _ln_in(x, s, b):
    m = jnp.mean(x, axis=-1, keepdims=True)
    v = jnp.mean((x - m) * (x - m), axis=-1, keepdims=True)
    return (x - m) * lax.rsqrt(v + 1e-6) * s + b


def _dot(a, b):
    return jnp.dot(a, b, preferred_element_type=jnp.float32)


def _dot_t(a, b):
    return lax.dot_general(a, b, (((1,), (1,)), ((), ())),
                           preferred_element_type=jnp.float32)


def _dot_tl(a, b):
    return lax.dot_general(a, b, (((0,), (0,)), ((), ())),
                           preferred_element_type=jnp.float32)


def _unit(u):
    l, r = divmod(u, NEXP * NCH)
    e, c = divmod(r, NCH)
    return l, e, c


def _mega_kernel(patches_ref, x_ref, dpc_ref, dps_ref, wp_ref,
                 dsc_ref, dss_ref, ws_ref, pos_ref,
                 ln1s_ref, ln1b_ref, ln2s_ref, ln2b_ref,
                 wqkv_ref, wo_ref, wr_ref, su_ref, sl_ref,
                 wcls_ref, bcls_ref, cat_ref, m_ref,
                 we1_hbm, we2_hbm,
                 out_ref,
                 h_s, y2b_s, o_s, pg_s, ps_s, xg_s, cov_s, flag_s,
                 w1buf, w2buf, sem1, sem2):
    bf16 = jnp.bfloat16

    def _issue(u):
        l, e, c = _unit(u)
        s = u % NBUF
        pltpu.make_async_copy(
            we1_hbm.at[l, e, :, pl.ds(c * HC, HC)], w1buf.at[s], sem1.at[s]
        ).start()
        pltpu.make_async_copy(
            we2_hbm.at[l, e, pl.ds(c * HC, HC), :], w2buf.at[s], sem2.at[s]
        ).start()

    def _wait(u):
        l, e, c = _unit(u)
        s = u % NBUF
        pltpu.make_async_copy(
            we1_hbm.at[l, e, :, pl.ds(c * HC, HC)], w1buf.at[s], sem1.at[s]
        ).wait()
        pltpu.make_async_copy(
            we2_hbm.at[l, e, pl.ds(c * HC, HC), :], w2buf.at[s], sem2.at[s]
        ).wait()

    for u in range(NBUF):
        _issue(u)

    pr = patches_ref[...]
    re = _dot(pr, dpc_ref[...])
    im = _dot(pr, dps_ref[...])
    pf = jnp.sqrt(re * re + im * im)
    tok = _dot(pf, wp_ref[...])
    xr = x_ref[...]
    sre = _dot(xr.astype(bf16), dsc_ref[...])
    sim_ = _dot(xr.astype(bf16), dss_ref[...])
    sf = jnp.sqrt(sre * sre + sim_ * sim_)
    cls = _dot(sf, ws_ref[...])
    pos = pos_ref[...]
    zero = jnp.zeros((LP - L, C), jnp.float32)
    for s in range(NSEQ):
        h_s[s * LP:s * LP + 1, :] = cls[s:s + 1, :] + pos[0:1, :]
        h_s[s * LP + 1:s * LP + L, :] = tok[s * NPATCH:(s + 1) * NPATCH, :] + pos[1:L, :]
        h_s[s * LP + L:(s + 1) * LP, :] = zero

    col_mask = lax.broadcasted_iota(jnp.int32, (LP, LP), 1)
    amask = jnp.where(col_mask < L, 0.0, -1e9).astype(jnp.float32)
    ecol = lax.broadcasted_iota(jnp.int32, (NTOK, 128), 1)
    rowv = (lax.broadcasted_iota(jnp.int32, (NTOK, 128), 0) % LP) < L
    gi = lax.broadcasted_iota(jnp.int32, (NTOK, SLOT), 1)

    unit_base = 0
    for l in range(LAYERS):
        hv = h_s[...]
        y = _ln_in(hv, ln1s_ref[l:l + 1, :], ln1b_ref[l:l + 1, :])
        qkv = _dot(y.astype(bf16), wqkv_ref[l]).astype(bf16)
        for s in range(NSEQ):
            r0 = s * LP
            qs = qkv[r0:r0 + LP, :]
            for hh in range(H):
                q = qs[:, hh * DH:(hh + 1) * DH]
                k = qs[:, C + hh * DH:C + (hh + 1) * DH]
                v = qs[:, 2 * C + hh * DH:2 * C + (hh + 1) * DH]
                sc = _dot_t(q, k) * (1.0 / np.sqrt(float(DH))) + amask
                mx = jnp.max(sc, axis=-1, keepdims=True)
                ex = jnp.exp(sc - mx)
                p = ex / jnp.sum(ex, axis=-1, keepdims=True)
                o_s[r0:r0 + LP, hh * DH:(hh + 1) * DH] = (
                    _dot(p.astype(bf16), v).astype(bf16))
        h_s[...] = hv + _dot(o_s[...], wo_ref[l])

        hv = h_s[...]
        y2 = _ln_in(hv, ln2s_ref[l:l + 1, :], ln2b_ref[l:l + 1, :])
        y2b_s[...] = y2.astype(bf16)
        logits = _dot(y2, wr_ref[l])
        logits = jnp.where(ecol < NEXP, logits, -1e30)
        mx = jnp.max(logits, axis=-1, keepdims=True)
        ex = jnp.exp(logits - mx)
        probs = ex / jnp.sum(ex, axis=-1, keepdims=True)
        su = su_ref[...]
        m1 = jnp.max(probs, axis=-1, keepdims=True)
        eq1 = (probs == m1).astype(jnp.float32)
        first = eq1 * (1.0 - jnp.minimum(_dot(eq1, su), 1.0))
        probs2 = probs - first * 2.0
        m2 = jnp.max(probs2, axis=-1, keepdims=True)
        eq2 = (probs2 == m2).astype(jnp.float32)
        second = eq2 * (1.0 - jnp.minimum(_dot(eq2, su), 1.0))
        denom = m1 + m2
        w1 = m1 / denom
        w2 = m2 / denom
        validf = rowv.astype(jnp.float32)
        first = first * validf
        second = second * validf
        sl = sl_ref[...]
        cnt1 = jnp.sum(first, axis=0, keepdims=True)
        r0r = _dot(sl, first.astype(bf16))
        r1r = cnt1 + _dot(sl, second.astype(bf16))
        basev = (ecol * CAP).astype(jnp.float32)
        ok0 = (r0r < CAP).astype(jnp.float32)
        ok1 = (r1r < CAP).astype(jnp.float32)
        sel0 = jnp.sum(first * ok0, axis=-1, keepdims=True)
        sel1 = jnp.sum(second * ok1, axis=-1, keepdims=True)
        pos0 = jnp.sum(first * ok0 * (basev + r0r), axis=-1, keepdims=True)
        pos1 = jnp.sum(second * ok1 * (basev + r1r), axis=-1, keepdims=True)
        big = float(SLOT + 7)
        pos0 = jnp.where(sel0 > 0.5, pos0, big)
        pos1 = jnp.where(sel1 > 0.5, pos1, big)
        hit0 = (gi == pos0.astype(jnp.int32))
        hit1 = (gi == pos1.astype(jnp.int32))
        pg_s[...] = (hit0 | hit1).astype(bf16)
        ps_s[...] = (hit0.astype(jnp.float32) * w1 +
                     hit1.astype(jnp.float32) * w2).astype(bf16)
        cov = first * (1.0 - ok0) * w1 + second * (1.0 - ok1) * w2
        cov_s[...] = cov
        covm = jnp.max(jnp.max(cov, axis=-1, keepdims=True),
                       axis=0, keepdims=True)
        flag_s[...] = covm + jnp.zeros((8, 128), jnp.float32)

        flag = flag_s[0, 0]
        for r in range(NEXP * NCH):
            u = unit_base + r
            _, e, c = _unit(u)
            _wait(u)
            su_ = u % NBUF
            w1c = w1buf[su_].astype(bf16)
            w2c = w2buf[su_].astype(bf16)
            if c == 0:
                xg_s[...] = _dot_tl(pg_s[:, e * CAP:(e + 1) * CAP],
                                    y2b_s[...]).astype(bf16)
            a = jax.nn.gelu(_dot(xg_s[...], w1c))
            o = _dot(a.astype(bf16), w2c)
            if u + NBUF < NUNIT:
                _issue(u + NBUF)
            h_s[...] += _dot(ps_s[:, e * CAP:(e + 1) * CAP], o.astype(bf16))

            @pl.when(flag > 0.0)
            def _():
                ad = jax.nn.gelu(_dot(y2b_s[...], w1c))
                od = _dot(ad.astype(bf16), w2c)
                h_s[...] += od * cov_s[:, e:e + 1]
        unit_base += NEXP * NCH

    clst = jnp.concatenate([h_s[s * LP:s * LP + 1, :] for s in range(NSEQ)],
                           axis=0)
    proj = _dot(clst.astype(bf16), wcls_ref[...]) + bcls_ref[...]
    pn = proj / (jnp.sqrt(jnp.sum(proj * proj, axis=-1, keepdims=True)) + 1e-8)
    ct = cat_ref[...]
    cn = ct / (jnp.sqrt(jnp.sum(ct * ct, axis=-1, keepdims=True)) + 1e-8)
    sim = _dot_t(pn, cn)
    out_ref[...] = _dot(m_ref[...], sim)


@jax.jit
def _run(x_enc, W_patch, W_seq, pos_emb, ln1_s, ln1_b, Wqkv, Wo,
         ln2_s, ln2_b, Wr, We1, We2, Wcls, bcls, cat_tok):
    f32 = jnp.float32
    bf16 = jnp.bfloat16
    xt = jnp.transpose(x_enc, (0, 2, 1)).reshape(NSEQ, T)
    patches = xt.reshape(NSEQ * NPATCH, P)
    pos_p = jnp.zeros((LP, C), f32).at[:L].set(pos_emb)
    wr_p = jnp.zeros((LAYERS, C, 128), f32).at[:, :, :NEXP].set(Wr)
    cat_p = jnp.zeros((16, C), f32).at[:K].set(cat_tok)
    mmat = np.zeros((8, 16), np.float32)
    for b in range(B):
        mmat[b, b * V:(b + 1) * V] = 1.0 / V

    vspec = pl.BlockSpec(memory_space=pltpu.VMEM)
    aspec = pl.BlockSpec(memory_space=pltpu.MemorySpace.HBM)
    out = pl.pallas_call(
        _mega_kernel,
        in_specs=[vspec] * 22 + [aspec, aspec],
        out_specs=vspec,
        out_shape=jax.ShapeDtypeStruct((8, 16), f32),
        scratch_shapes=[
            pltpu.VMEM((NTOK, C), f32),
            pltpu.VMEM((NTOK, C), bf16),
            pltpu.VMEM((NTOK, C), bf16),
            pltpu.VMEM((NTOK, SLOT), bf16),
            pltpu.VMEM((NTOK, SLOT), bf16),
            pltpu.VMEM((CAP, C), bf16),
            pltpu.VMEM((NTOK, 128), f32),
            pltpu.VMEM((8, 128), f32),
            pltpu.VMEM((NBUF, C, HC), f32),
            pltpu.VMEM((NBUF, HC, C), f32),
            pltpu.SemaphoreType.DMA((NBUF,)),
            pltpu.SemaphoreType.DMA((NBUF,)),
        ],
        compiler_params=pltpu.CompilerParams(
            vmem_limit_bytes=120 * 1024 * 1024,
        ),
    )(patches, xt, jnp.asarray(_DPC), jnp.asarray(_DPS), W_patch,
      jnp.asarray(_DSC).astype(bf16), jnp.asarray(_DSS).astype(bf16), W_seq,
      pos_p, ln1_s, ln1_b, ln2_s, ln2_b, Wqkv.astype(bf16), Wo.astype(bf16),
      wr_p, jnp.asarray(_SU128), jnp.asarray(_SL640).astype(bf16),
      Wcls.astype(bf16), bcls[None], cat_p, jnp.asarray(mmat),
      We1, We2)
    return out[:B, :K]


def kernel(x_enc, x_mark_enc, W_patch, W_seq, pos_emb, ln1_s, ln1_b, Wqkv, Wo,
           ln2_s, ln2_b, Wr, We1, We2, Wcls, bcls, cat_tok):
    return _run(x_enc, W_patch, W_seq, pos_emb, ln1_s, ln1_b, Wqkv, Wo,
                ln2_s, ln2_b, Wr, We1, We2, Wcls, bcls, cat_tok)
